# baseline (speedup 1.0000x reference)
import functools

import jax
import jax.numpy as jnp
from jax import lax
from jax.experimental import pallas as pl
from jax.experimental.pallas import tpu as pltpu
from jax.experimental.pallas import tpu_sc as plsc

VOCAB = 1000
CPAD = 1024
NTOK = 20480
NC, NS, L = 2, 16, 16
NW = NC * NS
TPW = NTOK // NW
CHUNK = 16
NCHUNK = TPW // CHUNK
TB = 128
NTB = NTOK // TB


def _sc_gather(idx3, table3):
    mesh = plsc.VectorSubcoreMesh(
        core_axis_name="c", subcore_axis_name="s", num_cores=NC, num_subcores=NS
    )

    @functools.partial(
        pl.kernel,
        mesh=mesh,
        compiler_params=pltpu.CompilerParams(use_tc_tiling_on_sc=True),
        out_type=jax.ShapeDtypeStruct((NTOK, 8, 128), jnp.float32),
        scratch_types=[
            pltpu.VMEM((TPW,), jnp.int32),
            pltpu.VMEM((CHUNK, 8, 128), jnp.float32),
            pltpu.VMEM((CHUNK, 8, 128), jnp.float32),
            pltpu.SemaphoreType.DMA,
            pltpu.SemaphoreType.DMA,
            pltpu.SemaphoreType.DMA,
            pltpu.SemaphoreType.DMA,
        ],
    )
    def k(idx_hbm, table_hbm, out_hbm, idx_v, buf0, buf1, gs0, gs1, os0, os1):
        wid = lax.axis_index("s") * NC + lax.axis_index("c")
        pltpu.sync_copy(idx_hbm.at[pl.ds(wid * TPW, TPW)], idx_v)
        bufs, gsems, osems = (buf0, buf1), (gs0, gs1), (os0, os1)

        def gather(c):
            b = c & 1
            return pltpu.make_async_copy(
                table_hbm.at[idx_v.at[pl.ds(c * CHUNK, CHUNK)]], bufs[b], gsems[b]
            )

        def put(c):
            b = c & 1
            return pltpu.make_async_copy(
                bufs[b], out_hbm.at[pl.ds(wid * TPW + c * CHUNK, CHUNK)], osems[b]
            )

        gather(0).start()
        for c in range(NCHUNK):
            if c + 1 < NCHUNK:
                if c >= 1:
                    put(c - 1).wait()
                gather(c + 1).start()
            gather(c).wait()
            put(c).start()
        put(NCHUNK - 2).wait()
        put(NCHUNK - 1).wait()

    return k(idx3, table3)


def _tc_loss_transpose(rows3, tgt3):
    def body(rows_ref, tgt_ref, out_ref, nll_ref):
        b = pl.program_id(0)
        x3 = rows_ref[...]
        h_iota = lax.broadcasted_iota(jnp.int32, (TB, 8, 128), 1)
        l_iota = lax.broadcasted_iota(jnp.int32, (TB, 8, 128), 2)
        valid = (h_iota * 128 + l_iota) < VOCAB
        neg = jnp.float32(-1e30)
        m = jnp.max(jnp.where(valid, x3, neg), axis=(1, 2), keepdims=True)
        e = jnp.where(valid, jnp.exp(x3 - m), 0.0)
        s = jnp.sum(e, axis=(1, 2), keepdims=True)
        lse = jnp.log(s) + m

        tgt = tgt_ref[0, :, :].reshape(TB, 1, 1)
        sel = (h_iota == tgt // 128) & (l_iota == tgt % 128)
        xt = jnp.sum(jnp.where(sel, x3, 0.0), axis=(1, 2), keepdims=True)
        blk = jnp.sum(lse - xt)

        @pl.when(b == 0)
        def _():
            nll_ref[...] = jnp.zeros((1, 1), jnp.float32)

        nll_ref[...] += blk.reshape(1, 1)

        for h in range(8):
            rows = 128 if h < 7 else VOCAB - 7 * 128
            y = jnp.transpose(x3[:, h, :], (1, 0))
            out_ref[pl.ds(h * 128, rows), :] = y[:rows, :]

    return pl.pallas_call(
        body,
        grid=(NTB,),
        in_specs=[
            pl.BlockSpec((TB, 8, 128), lambda b: (b, 0, 0)),
            pl.BlockSpec((1, TB, 1), lambda b: (b, 0, 0)),
        ],
        out_specs=[
            pl.BlockSpec((VOCAB, TB), lambda b: (0, b)),
            pl.BlockSpec((1, 1), lambda b: (0, 0)),
        ],
        out_shape=[
            jax.ShapeDtypeStruct((VOCAB, NTOK), jnp.float32),
            jax.ShapeDtypeStruct((1, 1), jnp.float32),
        ],
    )(rows3, tgt3)


def kernel(idx, targets, emb):
    idx3 = idx.reshape(NTOK)
    tgt3 = targets.reshape(NTB, TB, 1)
    table3 = jnp.pad(emb, ((0, 0), (0, CPAD - VOCAB))).reshape(VOCAB, 8, 128)
    rows3 = _sc_gather(idx3, table3)
    out2, nll = _tc_loss_transpose(rows3, tgt3)
    logits2 = out2.T
    loss = nll[0, 0] / NTOK
    return (logits2, loss)


# speedup vs baseline: 1.3424x; 1.3424x over previous
import functools

import jax
import jax.numpy as jnp
from jax import lax
from jax.experimental import pallas as pl
from jax.experimental.pallas import tpu as pltpu
from jax.experimental.pallas import tpu_sc as plsc

VOCAB = 1000
CPAD = 1024
NTOK = 20480
NC, NS, L = 2, 16, 16
NW = NC * NS
TPW = NTOK // NW
CHUNK = 16
NCHUNK = TPW // CHUNK
TB = 128
NTB = NTOK // TB


def _sc_gather(idx3, table3):
    mesh = plsc.VectorSubcoreMesh(
        core_axis_name="c", subcore_axis_name="s", num_cores=NC, num_subcores=NS
    )

    @functools.partial(
        pl.kernel,
        mesh=mesh,
        compiler_params=pltpu.CompilerParams(use_tc_tiling_on_sc=True),
        out_type=jax.ShapeDtypeStruct((NTOK, 8, 128), jnp.float32),
        scratch_types=[
            pltpu.VMEM((TPW,), jnp.int32),
            pltpu.VMEM((CHUNK, 8, 128), jnp.float32),
            pltpu.VMEM((CHUNK, 8, 128), jnp.float32),
            pltpu.SemaphoreType.DMA,
            pltpu.SemaphoreType.DMA,
            pltpu.SemaphoreType.DMA,
            pltpu.SemaphoreType.DMA,
        ],
    )
    def k(idx_hbm, table_hbm, out_hbm, idx_v, buf0, buf1, gs0, gs1, os0, os1):
        wid = lax.axis_index("s") * NC + lax.axis_index("c")
        pltpu.sync_copy(idx_hbm.at[pl.ds(wid * TPW, TPW)], idx_v)
        bufs, gsems, osems = (buf0, buf1), (gs0, gs1), (os0, os1)

        def gather(c):
            b = c & 1
            return pltpu.make_async_copy(
                table_hbm.at[idx_v.at[pl.ds(c * CHUNK, CHUNK)]], bufs[b], gsems[b]
            )

        def put(c):
            b = c & 1
            return pltpu.make_async_copy(
                bufs[b], out_hbm.at[pl.ds(wid * TPW + c * CHUNK, CHUNK)], osems[b]
            )

        gather(0).start()
        for c in range(NCHUNK):
            if c + 1 < NCHUNK:
                if c >= 1:
                    put(c - 1).wait()
                gather(c + 1).start()
            gather(c).wait()
            put(c).start()
        put(NCHUNK - 2).wait()
        put(NCHUNK - 1).wait()

    return k(idx3, table3)


def _tc_loss_transpose(rows3, tgt3):
    def body(rows_ref, tgt_ref, out_ref, nll_ref):
        b = pl.program_id(0)
        x3 = rows_ref[...]

        for h in range(8):
            rows = 128 if h < 7 else VOCAB - 7 * 128
            y = jnp.transpose(x3[:, h, :], (1, 0))
            out_ref[pl.ds(h * 128, rows), :] = y[:rows, :]

        y = out_ref[...]
        m = jnp.max(y, axis=0, keepdims=True)
        s = jnp.sum(jnp.exp(y - m), axis=0, keepdims=True)
        lse = jnp.log(s) + m

        tgt = tgt_ref[0, :, :]
        riota = lax.broadcasted_iota(jnp.int32, (VOCAB, TB), 0)
        xt = jnp.sum(jnp.where(riota == tgt, y, 0.0), axis=0, keepdims=True)
        blk = jnp.sum(lse - xt)

        @pl.when(b == 0)
        def _():
            nll_ref[...] = jnp.zeros((1, 1), jnp.float32)

        nll_ref[...] += blk.reshape(1, 1)

    return pl.pallas_call(
        body,
        grid=(NTB,),
        in_specs=[
            pl.BlockSpec((TB, 8, 128), lambda b: (b, 0, 0)),
            pl.BlockSpec((1, 1, TB), lambda b: (b, 0, 0)),
        ],
        out_specs=[
            pl.BlockSpec((VOCAB, TB), lambda b: (0, b)),
            pl.BlockSpec((1, 1), lambda b: (0, 0)),
        ],
        out_shape=[
            jax.ShapeDtypeStruct((VOCAB, NTOK), jnp.float32),
            jax.ShapeDtypeStruct((1, 1), jnp.float32),
        ],
    )(rows3, tgt3)


def kernel(idx, targets, emb):
    idx3 = idx.reshape(NTOK)
    tgt3 = targets.reshape(NTB, 1, TB)
    table3 = jnp.pad(emb, ((0, 0), (0, CPAD - VOCAB))).reshape(VOCAB, 8, 128)
    rows3 = _sc_gather(idx3, table3)
    out2, nll = _tc_loss_transpose(rows3, tgt3)
    logits2 = out2.T
    loss = nll[0, 0] / NTOK
    return (logits2, loss)
